# Initial kernel scaffold; baseline (speedup 1.0000x reference)
import jax
import jax.numpy as jnp
from jax.experimental import pallas as pl


def kernel(x, edge_index, batch, Wl1, bl1, Wr1, Wl2, bl2, Wr2, Wl3, bl3, Wr3, Wc1, bc1, Wc2, bc2, Wc3, bc3):
    raise NotImplementedError("write your pallas kernel here")


# speedup vs baseline: 10.0076x; 10.0076x over previous
import functools

import jax
import jax.numpy as jnp
from jax import lax
from jax.experimental import pallas as pl
from jax.experimental.pallas import tpu as pltpu
from jax.experimental.pallas import tpu_sc as plsc

N = 10000
E = 320000
IN = 128
H = 64
G = 64

NC = 2
NS = 16
NW = NC * NS
EPW = E // NW
BLK = 80
NBUF = 5
NBLK = EPW // BLK
NOUT = NBLK // NBUF
NP = 10240
RPW = NP // NS
CH = 64
HF = 4 * H


def _leaky(h):
    return jnp.where(h > 0, h, 0.2 * h)


def _sc_mesh():
    return plsc.VectorSubcoreMesh(core_axis_name="c", subcore_axis_name="s")


def _make_agg(C, with_deg):
    out_type = [jax.ShapeDtypeStruct((NP, C), jnp.float32) for _ in range(NC)]
    if with_deg:
        out_type += [jax.ShapeDtypeStruct((NP, 16), jnp.float32) for _ in range(NC)]

    scratch = [
        pltpu.VMEM((EPW,), jnp.int32),
        pltpu.VMEM((NBLK, BLK), jnp.int32),
        pltpu.VMEM((NBUF, BLK, C), jnp.float32),
        pltpu.VMEM_SHARED((NP, C), jnp.float32),
        pltpu.SemaphoreType.DMA,
        pltpu.SemaphoreType.DMA,
    ]
    if with_deg:
        scratch += [
            pltpu.VMEM((BLK, 16), jnp.float32),
            pltpu.VMEM_SHARED((NP, 16), jnp.float32),
        ]

    def body(table, srcr, dstr, zrows, zdeg, ones, *rest):
        if with_deg:
            out0, out1, deg0, deg1 = rest[:4]
            src_v, dst_v, rows_v, acc, gsem, ssem, ones_v, dacc = rest[4:]
        else:
            out0, out1 = rest[:2]
            src_v, dst_v, rows_v, acc, gsem, ssem = rest[2:]

        c = lax.axis_index("c")
        s = lax.axis_index("s")
        w = c * NS + s

        pltpu.sync_copy(zrows, acc.at[pl.ds(s * RPW, RPW)])
        if with_deg:
            pltpu.sync_copy(zdeg, dacc.at[pl.ds(s * RPW, RPW)])
            pltpu.sync_copy(ones, ones_v)

        pltpu.sync_copy(srcr.at[pl.ds(w * EPW, EPW)], src_v)
        pltpu.sync_copy(dstr.at[w], dst_v)
        plsc.subcore_barrier()

        @pl.loop(0, NOUT)
        def _(jj):
            j0 = jj * NBUF
            gds = []
            for b in range(NBUF):
                idx = src_v.at[pl.ds((j0 + b) * BLK, BLK)]
                gds.append(pltpu.async_copy(table.at[idx], rows_v.at[b], gsem))
            sds = []
            if with_deg:
                for b in range(NBUF):
                    sds.append(pltpu.async_copy(
                        ones_v, dacc.at[dst_v.at[j0 + b]], ssem, add=True))
            for d in gds:
                d.wait()
            for b in range(NBUF):
                sds.append(pltpu.async_copy(
                    rows_v.at[b], acc.at[dst_v.at[j0 + b]], ssem, add=True))
            for d in sds:
                d.wait()

        plsc.subcore_barrier()

        rows = pl.ds(s * RPW, RPW)

        @pl.when(c == 0)
        def _():
            pltpu.sync_copy(acc.at[rows], out0.at[rows])
            if with_deg:
                pltpu.sync_copy(dacc.at[rows], deg0.at[rows])

        @pl.when(c == 1)
        def _():
            pltpu.sync_copy(acc.at[rows], out1.at[rows])
            if with_deg:
                pltpu.sync_copy(dacc.at[rows], deg1.at[rows])

    return pl.kernel(body, out_type=out_type, mesh=_sc_mesh(),
                     scratch_types=scratch,
                     compiler_params=pltpu.CompilerParams(
                         use_tc_tiling_on_sc=False))


EPW2 = E // NS
NBLK2 = EPW2 // BLK
NOUT2 = NBLK2 // NBUF


def _make_agg3():
    out_type = [jax.ShapeDtypeStruct((NP, H), jnp.float32) for _ in range(NC)]
    scratch = [
        pltpu.VMEM((EPW2,), jnp.int32),
        pltpu.VMEM((NBLK2, BLK), jnp.int32),
        pltpu.VMEM((NBUF, BLK, H), jnp.float32),
        pltpu.VMEM_SHARED((NP, H), jnp.float32),
        pltpu.SemaphoreType.DMA,
        pltpu.SemaphoreType.DMA,
    ]

    def body(ta, tb, srcr, dstr, zrows, out0, out1,
             src_v, dst_v, rows_v, acc, gsem, ssem):
        c = lax.axis_index("c")
        s = lax.axis_index("s")

        pltpu.sync_copy(zrows, acc.at[pl.ds(s * RPW, RPW)])
        pltpu.sync_copy(srcr.at[pl.ds(s * EPW2, EPW2)], src_v)
        pltpu.sync_copy(dstr.at[s], dst_v)
        plsc.subcore_barrier()

        def run(table):
            @pl.loop(0, NOUT2)
            def _(jj):
                j0 = jj * NBUF
                gds = []
                for b in range(NBUF):
                    idx = src_v.at[pl.ds((j0 + b) * BLK, BLK)]
                    gds.append(pltpu.async_copy(table.at[idx],
                                                rows_v.at[b], gsem))
                for d in gds:
                    d.wait()
                sds = []
                for b in range(NBUF):
                    sds.append(pltpu.async_copy(
                        rows_v.at[b], acc.at[dst_v.at[j0 + b]],
                        ssem, add=True))
                for d in sds:
                    d.wait()

        @pl.when(c == 0)
        def _():
            run(ta)

        @pl.when(c == 1)
        def _():
            run(tb)

        plsc.subcore_barrier()
        rows = pl.ds(s * RPW, RPW)

        @pl.when(c == 0)
        def _():
            pltpu.sync_copy(acc.at[rows], out0.at[rows])

        @pl.when(c == 1)
        def _():
            pltpu.sync_copy(acc.at[rows], out1.at[rows])

    return pl.kernel(body, out_type=out_type, mesh=_sc_mesh(),
                     scratch_types=scratch,
                     compiler_params=pltpu.CompilerParams(
                         use_tc_tiling_on_sc=False))


def _pool_body(h3p, batch, hg, cnts, batch_v, rowbuf, outbuf, cntbuf):
    c = lax.axis_index("c")
    s = lax.axis_index("s")
    w = c * NS + s
    g0 = 2 * w
    g1 = 2 * w + 1

    pltpu.sync_copy(batch, batch_v)

    zi = jnp.zeros((16,), jnp.int32)

    def cnt_body(i, carry):
        nlt, c0, c1 = carry
        v = batch_v[pl.ds(i * 16, 16)]
        nlt = nlt + plsc.all_reduce_population_count(v < g0)
        c0 = c0 + plsc.all_reduce_population_count(v == g0)
        c1 = c1 + plsc.all_reduce_population_count(v == g1)
        return nlt, c0, c1

    nlt, c0, c1 = lax.fori_loop(0, NP // 16, cnt_body, (zi, zi, zi))
    start0 = jnp.max(nlt, axis=0)
    cnt0 = jnp.max(c0, axis=0)
    cnt1 = jnp.max(c1, axis=0)
    start1 = start0 + cnt0

    NV = HF // 16
    zf = jnp.zeros((16,), jnp.float32)
    ninf = jnp.full((16,), -3.0e38, jnp.float32)

    def row_add(r, carry):
        acc = list(carry)
        for q in range(NV):
            val = rowbuf[r, pl.ds(q * 16, 16)]
            acc[q] = acc[q] + val
            acc[NV + q] = jnp.maximum(acc[NV + q], val)
        return tuple(acc)

    def row_add_masked(ntail):
        def f(r, carry):
            acc = list(carry)
            valid = r < ntail
            for q in range(NV):
                val = rowbuf[r, pl.ds(q * 16, 16)]
                acc[q] = acc[q] + jnp.where(valid, val, 0.0)
                acc[NV + q] = jnp.maximum(
                    acc[NV + q], jnp.where(valid, val, -3.0e38))
            return tuple(acc)
        return f

    def seg_reduce(start, cnt):
        init = tuple([zf] * NV + [ninf] * NV)
        nfull = cnt // CH

        def chunk(k, carry):
            pltpu.sync_copy(h3p.at[pl.ds(start + k * CH, CH)], rowbuf)
            return lax.fori_loop(0, CH, row_add, carry)

        carry = lax.fori_loop(0, nfull, chunk, init)
        ntail = cnt - nfull * CH
        pltpu.sync_copy(h3p.at[pl.ds(start + nfull * CH, CH)], rowbuf)
        carry = lax.fori_loop(0, CH, row_add_masked(ntail), carry)
        return carry

    res0 = seg_reduce(start0, cnt0)
    res1 = seg_reduce(start1, cnt1)

    for row, res, cv in ((0, res0, c0), (1, res1, c1)):
        nonempty = cv > zi
        for q in range(NV):
            outbuf[row, pl.ds(q * 16, 16)] = res[q]
            outbuf[row, pl.ds(HF + q * 16, 16)] = jnp.where(
                nonempty, res[NV + q], 0.0)
        cntbuf[row, :] = cv.astype(jnp.float32)

    pltpu.sync_copy(outbuf, hg.at[pl.ds(2 * w, 2)])
    pltpu.sync_copy(cntbuf, cnts.at[pl.ds(2 * w, 2)])


_pool = pl.kernel(
    _pool_body,
    out_type=[jax.ShapeDtypeStruct((G, 2 * HF), jnp.float32),
              jax.ShapeDtypeStruct((G, 16), jnp.float32)],
    mesh=_sc_mesh(),
    scratch_types=[
        pltpu.VMEM((NP,), jnp.int32),
        pltpu.VMEM((CH, HF), jnp.float32),
        pltpu.VMEM((2, 2 * HF), jnp.float32),
        pltpu.VMEM((2, 16), jnp.float32),
    ],
    compiler_params=pltpu.CompilerParams(use_tc_tiling_on_sc=False,
                                         needs_layout_passes=False),
)


RB = 1024
NRB = NP // RB


def _t1_body(x_ref, w_ref, outl, outr):
    acc = jnp.dot(x_ref[:], w_ref[:], preferred_element_type=jnp.float32)
    outl[:] = acc[:, :H]
    outr[:] = acc[:, H:]


def _t1(x, wcat):
    return pl.pallas_call(
        _t1_body,
        grid=(NRB,),
        in_specs=[
            pl.BlockSpec((RB, IN), lambda i: (i, 0)),
            pl.BlockSpec((IN, 2 * H), lambda i: (0, 0)),
        ],
        out_specs=[
            pl.BlockSpec((RB, H), lambda i: (i, 0)),
            pl.BlockSpec((RB, H), lambda i: (i, 0)),
        ],
        out_shape=[
            jax.ShapeDtypeStruct((NP, H), jnp.float32),
            jax.ShapeDtypeStruct((NP, H), jnp.float32),
        ],
    )(x, wcat)


def _deginv(d0, d1):
    deg = d0[:, 0:1] + d1[:, 0:1]
    return 1.0 / jnp.maximum(deg, 1.0)


def _t2_body(a0, a1, d0, d1, xr1, bl1, w2, bl2, h1_out, hr2_out):
    mean = (a0[:] + a1[:]) * _deginv(d0[:], d1[:])
    h1 = _leaky(mean + bl1[:] + xr1[:])
    h1_out[:] = h1
    hr2_out[:] = jnp.dot(h1, w2[:], preferred_element_type=jnp.float32) + bl2[:]


def _t2(a0, a1, d0, d1, xr1, bl1, w2, bl2):
    row = lambda i: (i, 0)
    zero = lambda i: (0, 0)
    return pl.pallas_call(
        _t2_body,
        grid=(NRB,),
        in_specs=[
            pl.BlockSpec((RB, H), row),
            pl.BlockSpec((RB, H), row),
            pl.BlockSpec((RB, 16), row),
            pl.BlockSpec((RB, 16), row),
            pl.BlockSpec((RB, H), row),
            pl.BlockSpec((1, H), zero),
            pl.BlockSpec((H, 2 * H), zero),
            pl.BlockSpec((1, 2 * H), zero),
        ],
        out_specs=[
            pl.BlockSpec((RB, H), row),
            pl.BlockSpec((RB, 2 * H), row),
        ],
        out_shape=[
            jax.ShapeDtypeStruct((NP, H), jnp.float32),
            jax.ShapeDtypeStruct((NP, 2 * H), jnp.float32),
        ],
    )(a0, a1, d0, d1, xr1, bl1, w2, bl2)


def _t3_body(a0, a1, d0, d1, hr2, wl2, wr3, bl3, h2a_out, h2b_out, hr3_out):
    mean = (a0[:] + a1[:]) * _deginv(d0[:], d1[:])
    h2 = _leaky(jnp.dot(mean, wl2[:], preferred_element_type=jnp.float32)
                + hr2[:])
    h2a_out[:] = h2[:, :H]
    h2b_out[:] = h2[:, H:]
    hr3_out[:] = jnp.dot(h2, wr3[:], preferred_element_type=jnp.float32) + bl3[:]


def _t3(a0, a1, d0, d1, hr2, wl2, wr3, bl3):
    row = lambda i: (i, 0)
    zero = lambda i: (0, 0)
    return pl.pallas_call(
        _t3_body,
        grid=(NRB,),
        in_specs=[
            pl.BlockSpec((RB, H), row),
            pl.BlockSpec((RB, H), row),
            pl.BlockSpec((RB, 16), row),
            pl.BlockSpec((RB, 16), row),
            pl.BlockSpec((RB, 2 * H), row),
            pl.BlockSpec((H, 2 * H), zero),
            pl.BlockSpec((2 * H, 4 * H), zero),
            pl.BlockSpec((1, 4 * H), zero),
        ],
        out_specs=[
            pl.BlockSpec((RB, H), row),
            pl.BlockSpec((RB, H), row),
            pl.BlockSpec((RB, 4 * H), row),
        ],
        out_shape=[
            jax.ShapeDtypeStruct((NP, H), jnp.float32),
            jax.ShapeDtypeStruct((NP, H), jnp.float32),
            jax.ShapeDtypeStruct((NP, 4 * H), jnp.float32),
        ],
    )(a0, a1, d0, d1, hr2, wl2, wr3, bl3)


def _t4_body(a0, a1, d0, d1, hr3, wl3, h3_out):
    mean = jnp.concatenate([a0[:], a1[:]], axis=1) * _deginv(d0[:], d1[:])
    h3_out[:] = _leaky(
        jnp.dot(mean, wl3[:], preferred_element_type=jnp.float32) + hr3[:])


def _t4(a0, a1, d0, d1, hr3, wl3):
    row = lambda i: (i, 0)
    zero = lambda i: (0, 0)
    return pl.pallas_call(
        _t4_body,
        grid=(NRB,),
        in_specs=[
            pl.BlockSpec((RB, H), row),
            pl.BlockSpec((RB, H), row),
            pl.BlockSpec((RB, 16), row),
            pl.BlockSpec((RB, 16), row),
            pl.BlockSpec((RB, 4 * H), row),
            pl.BlockSpec((2 * H, 4 * H), zero),
        ],
        out_specs=pl.BlockSpec((RB, 4 * H), row),
        out_shape=jax.ShapeDtypeStruct((NP, 4 * H), jnp.float32),
    )(a0, a1, d0, d1, hr3, wl3)


def _t5_body(hg, cnts, wc1, bc1, wc2, bc2, wc3, bc3, out):
    inv = 1.0 / jnp.maximum(cnts[:, 0:1], 1.0)
    hgf = jnp.concatenate([hg[:, :HF] * inv, hg[:, HF:]], axis=1)
    z = _leaky(jnp.dot(hgf, wc1[:], preferred_element_type=jnp.float32)
               + bc1[:])
    z = _leaky(jnp.dot(z, wc2[:], preferred_element_type=jnp.float32)
               + bc2[:])
    out[:] = jnp.dot(z, wc3[:], preferred_element_type=jnp.float32) + bc3[:]


def _t5(hg, cnts, wc1, bc1, wc2, bc2, wc3p, bc3p):
    return pl.pallas_call(
        _t5_body,
        out_shape=jax.ShapeDtypeStruct((G, 128), jnp.float32),
    )(hg, cnts, wc1, bc1, wc2, bc2, wc3p, bc3p)


_agg_deg = _make_agg(H, True)
_agg_h = _make_agg(H, False)
_agg3 = _make_agg3()


def kernel(x, edge_index, batch, Wl1, bl1, Wr1, Wl2, bl2, Wr2,
           Wl3, bl3, Wr3, Wc1, bc1, Wc2, bc2, Wc3, bc3):
    x = jnp.pad(x, ((0, NP - N), (0, 0)))
    src = edge_index[0].astype(jnp.int32)
    dst_flat = edge_index[1].astype(jnp.int32)
    dst = dst_flat.reshape(NW, NBLK, BLK)
    dst3 = dst_flat.reshape(NS, NBLK2, BLK)
    batch = jnp.pad(batch.astype(jnp.int32), (0, NP - N), constant_values=G)

    z64 = jnp.zeros((RPW, H), jnp.float32)
    z16 = jnp.zeros((RPW, 16), jnp.float32)
    ones = jnp.ones((BLK, 16), jnp.float32)

    xl1, xr1 = _t1(x, jnp.concatenate([Wl1, Wr1], axis=1))
    a0, a1, d0, d1 = _agg_deg(xl1, src, dst, z64, z16, ones)
    h1, hr2 = _t2(a0, a1, d0, d1, xr1, bl1.reshape(1, H),
                  Wr2, bl2.reshape(1, 2 * H))

    b0, b1 = _agg_h(h1, src, dst, z64, z16, ones)
    h2a, h2b, hr3 = _t3(b0, b1, d0, d1, hr2, Wl2, Wr3, bl3.reshape(1, 4 * H))

    c0, c1 = _agg3(h2a, h2b, src, dst3, z64)
    h3 = _t4(c0, c1, d0, d1, hr3, Wl3)

    hg, cnts = _pool(h3, batch)
    wc3p = jnp.pad(Wc3, ((0, 0), (0, 127)))
    bc3p = jnp.pad(bc3, (0, 127)).reshape(1, 128)
    out = _t5(hg, cnts, Wc1, bc1.reshape(1, 2 * H), Wc2, bc2.reshape(1, H),
              wc3p, bc3p)
    return out[:, 0]


# speedup vs baseline: 11.1079x; 1.1099x over previous
import functools

import jax
import jax.numpy as jnp
from jax import lax
from jax.experimental import pallas as pl
from jax.experimental.pallas import tpu as pltpu
from jax.experimental.pallas import tpu_sc as plsc

N = 10000
E = 320000
IN = 128
H = 64
G = 64

NC = 2
NS = 16
NW = NC * NS
EPW = E // NW
BLK = 80
NBUF = 5
NBLK = EPW // BLK
NOUT = NBLK // NBUF
RPW = N // NS
CH = 64
HF = 4 * H


def _leaky(h):
    return jnp.where(h > 0, h, 0.2 * h)


def _sc_mesh():
    return plsc.VectorSubcoreMesh(core_axis_name="c", subcore_axis_name="s")


def _edge_pipeline(table, src_v, dst_v, rows_v, acc, gsems, ssems, nout):
    def fire_g(it, hs):
        for b in range(NBUF):
            idx = src_v.at[pl.ds((it * NBUF + b) * BLK, BLK)]
            pltpu.async_copy(table.at[idx], rows_v.at[hs].at[b], gsems[hs])

    def fire_s(it, hs):
        for b in range(NBUF):
            pltpu.async_copy(rows_v.at[hs].at[b],
                             acc.at[dst_v.at[it * NBUF + b]],
                             ssems[hs], add=True)

    def wait_g(hs):
        for b in range(NBUF):
            idx = src_v.at[pl.ds(b * BLK, BLK)]
            pltpu.make_async_copy(table.at[idx], rows_v.at[hs].at[b],
                                  gsems[hs]).wait()

    def wait_s(hs):
        for b in range(NBUF):
            pltpu.make_async_copy(rows_v.at[hs].at[b], acc.at[dst_v.at[b]],
                                  ssems[hs]).wait()

    P = (nout - 1) // 2
    fire_g(0, 0)

    @pl.loop(0, P)
    def _(p):
        itA = 2 * p
        wait_g(0)
        fire_s(itA, 0)

        @pl.when(p > 0)
        def _():
            wait_s(1)

        fire_g(itA + 1, 1)
        wait_g(1)
        fire_s(itA + 1, 1)
        wait_s(0)
        fire_g(itA + 2, 0)

    wait_g(0)
    fire_s(2 * P, 0)
    if nout % 2 == 1:
        wait_s(1)
        wait_s(0)
    else:
        wait_s(1)
        fire_g(nout - 1, 1)
        wait_g(1)
        fire_s(nout - 1, 1)
        wait_s(0)
        wait_s(1)


def _make_agg(C):
    out_type = [jax.ShapeDtypeStruct((N, C), jnp.float32) for _ in range(NC)]

    scratch = [
        pltpu.VMEM((EPW,), jnp.int32),
        pltpu.VMEM((NBLK, BLK), jnp.int32),
        pltpu.VMEM((2, NBUF, BLK, C), jnp.float32),
        pltpu.VMEM_SHARED((N, C), jnp.float32),
        pltpu.SemaphoreType.DMA,
        pltpu.SemaphoreType.DMA,
        pltpu.SemaphoreType.DMA,
        pltpu.SemaphoreType.DMA,
    ]

    def body(table, srcr, dstr, zrows, out0, out1,
             src_v, dst_v, rows_v, acc, gsemA, gsemB, ssemA, ssemB):
        c = lax.axis_index("c")
        s = lax.axis_index("s")
        w = c * NS + s

        pltpu.sync_copy(zrows, acc.at[pl.ds(s * RPW, RPW)])

        pltpu.sync_copy(srcr.at[pl.ds(w * EPW, EPW)], src_v)
        pltpu.sync_copy(dstr.at[w], dst_v)
        plsc.subcore_barrier()

        _edge_pipeline(table, src_v, dst_v, rows_v, acc,
                       (gsemA, gsemB), (ssemA, ssemB), NOUT)

        plsc.subcore_barrier()

        rows = pl.ds(s * RPW, RPW)

        @pl.when(c == 0)
        def _():
            pltpu.sync_copy(acc.at[rows], out0.at[rows])

        @pl.when(c == 1)
        def _():
            pltpu.sync_copy(acc.at[rows], out1.at[rows])

    return pl.kernel(body, out_type=out_type, mesh=_sc_mesh(),
                     scratch_types=scratch,
                     compiler_params=pltpu.CompilerParams(
                         use_tc_tiling_on_sc=False))


def _make_deg():
    out_type = [jax.ShapeDtypeStruct((N, 8), jnp.float32) for _ in range(NC)]
    scratch = [
        pltpu.VMEM((NBLK, BLK), jnp.int32),
        pltpu.VMEM((BLK, 8), jnp.float32),
        pltpu.VMEM_SHARED((N, 8), jnp.float32),
        pltpu.SemaphoreType.DMA,
    ]

    def body(dstr, zdeg, ones, deg0, deg1, dst_v, ones_v, dacc, dsem):
        c = lax.axis_index("c")
        s = lax.axis_index("s")
        w = c * NS + s

        pltpu.sync_copy(zdeg, dacc.at[pl.ds(s * RPW, RPW)])
        pltpu.sync_copy(ones, ones_v)
        pltpu.sync_copy(dstr.at[w], dst_v)
        plsc.subcore_barrier()

        @pl.loop(0, NBLK)
        def _(j):
            pltpu.async_copy(ones_v, dacc.at[dst_v.at[j]], dsem, add=True)

        @pl.loop(0, NBLK)
        def _(j):
            pltpu.make_async_copy(ones_v, dacc.at[dst_v.at[0]], dsem).wait()

        plsc.subcore_barrier()
        rows = pl.ds(s * RPW, RPW)

        @pl.when(c == 0)
        def _():
            pltpu.sync_copy(dacc.at[rows], deg0.at[rows])

        @pl.when(c == 1)
        def _():
            pltpu.sync_copy(dacc.at[rows], deg1.at[rows])

    return pl.kernel(body, out_type=out_type, mesh=_sc_mesh(),
                     scratch_types=scratch,
                     compiler_params=pltpu.CompilerParams(
                         use_tc_tiling_on_sc=False))


def _make_deg():
    out_type = [jax.ShapeDtypeStruct((N, 8), jnp.float32) for _ in range(NC)]
    scratch = [
        pltpu.VMEM((NBLK, BLK), jnp.int32),
        pltpu.VMEM((BLK, 8), jnp.float32),
        pltpu.VMEM_SHARED((N, 8), jnp.float32),
        pltpu.SemaphoreType.DMA,
    ]

    def body(dstr, zdeg, ones, deg0, deg1, dst_v, ones_v, dacc, dsem):
        c = lax.axis_index("c")
        s = lax.axis_index("s")
        w = c * NS + s

        pltpu.sync_copy(zdeg, dacc.at[pl.ds(s * RPW, RPW)])
        pltpu.sync_copy(ones, ones_v)
        pltpu.sync_copy(dstr.at[w], dst_v)
        plsc.subcore_barrier()

        @pl.loop(0, NBLK)
        def _(j):
            pltpu.async_copy(ones_v, dacc.at[dst_v.at[j]], dsem, add=True)

        @pl.loop(0, NBLK)
        def _(j):
            pltpu.make_async_copy(ones_v, dacc.at[dst_v.at[0]], dsem).wait()

        plsc.subcore_barrier()
        rows = pl.ds(s * RPW, RPW)

        @pl.when(c == 0)
        def _():
            pltpu.sync_copy(dacc.at[rows], deg0.at[rows])

        @pl.when(c == 1)
        def _():
            pltpu.sync_copy(dacc.at[rows], deg1.at[rows])

    return pl.kernel(body, out_type=out_type, mesh=_sc_mesh(),
                     scratch_types=scratch,
                     compiler_params=pltpu.CompilerParams(
                         use_tc_tiling_on_sc=False))


EPW2 = E // NS
NBLK2 = EPW2 // BLK
NOUT2 = NBLK2 // NBUF


def _make_fsplit(CHALF):
    out_type = [jax.ShapeDtypeStruct((N, CHALF), jnp.float32) for _ in range(NC)]
    scratch = [
        pltpu.VMEM((EPW2,), jnp.int32),
        pltpu.VMEM((NBLK2, BLK), jnp.int32),
        pltpu.VMEM((2, NBUF, BLK, CHALF), jnp.float32),
        pltpu.VMEM_SHARED((N, CHALF), jnp.float32),
        pltpu.SemaphoreType.DMA,
        pltpu.SemaphoreType.DMA,
        pltpu.SemaphoreType.DMA,
        pltpu.SemaphoreType.DMA,
    ]

    def body(ta, tb, srcr, dstr, zrows, out0, out1,
             src_v, dst_v, rows_v, acc, gsemA, gsemB, ssemA, ssemB):
        c = lax.axis_index("c")
        s = lax.axis_index("s")

        pltpu.sync_copy(zrows, acc.at[pl.ds(s * RPW, RPW)])
        pltpu.sync_copy(srcr.at[pl.ds(s * EPW2, EPW2)], src_v)
        pltpu.sync_copy(dstr.at[s], dst_v)
        plsc.subcore_barrier()

        def run(table):
            _edge_pipeline(table, src_v, dst_v, rows_v, acc,
                           (gsemA, gsemB), (ssemA, ssemB), NOUT2)

        @pl.when(c == 0)
        def _():
            run(ta)

        @pl.when(c == 1)
        def _():
            run(tb)

        plsc.subcore_barrier()
        rows = pl.ds(s * RPW, RPW)

        @pl.when(c == 0)
        def _():
            pltpu.sync_copy(acc.at[rows], out0.at[rows])

        @pl.when(c == 1)
        def _():
            pltpu.sync_copy(acc.at[rows], out1.at[rows])

    return pl.kernel(body, out_type=out_type, mesh=_sc_mesh(),
                     scratch_types=scratch,
                     compiler_params=pltpu.CompilerParams(
                         use_tc_tiling_on_sc=False))


def _pool_body(h3p, batch, hg, cnts, batch_v, rowbuf, outbuf, cntbuf):
    c = lax.axis_index("c")
    s = lax.axis_index("s")
    w = c * NS + s
    g0 = 2 * w
    g1 = 2 * w + 1

    pltpu.sync_copy(batch, batch_v)

    zi = jnp.zeros((16,), jnp.int32)

    def cnt_body(i, carry):
        nlt, c0, c1 = carry
        v = batch_v[pl.ds(i * 16, 16)]
        nlt = nlt + plsc.all_reduce_population_count(v < g0)
        c0 = c0 + plsc.all_reduce_population_count(v == g0)
        c1 = c1 + plsc.all_reduce_population_count(v == g1)
        return nlt, c0, c1

    nlt, c0, c1 = lax.fori_loop(0, N // 16, cnt_body, (zi, zi, zi))
    start0 = jnp.max(nlt, axis=0)
    cnt0 = jnp.max(c0, axis=0)
    cnt1 = jnp.max(c1, axis=0)
    start1 = start0 + cnt0

    NV = HF // 16
    zf = jnp.zeros((16,), jnp.float32)
    ninf = jnp.full((16,), -3.0e38, jnp.float32)

    def row_add(r, carry):
        acc = list(carry)
        for q in range(NV):
            val = rowbuf[r, pl.ds(q * 16, 16)]
            acc[q] = acc[q] + val
            acc[NV + q] = jnp.maximum(acc[NV + q], val)
        return tuple(acc)

    def row_add_masked(lo, hi):
        def f(r, carry):
            acc = list(carry)
            valid = (r >= lo) & (r < hi)
            for q in range(NV):
                val = rowbuf[r, pl.ds(q * 16, 16)]
                acc[q] = acc[q] + jnp.where(valid, val, 0.0)
                acc[NV + q] = jnp.maximum(
                    acc[NV + q], jnp.where(valid, val, -3.0e38))
            return tuple(acc)
        return f

    def seg_reduce(start, cnt):
        init = tuple([zf] * NV + [ninf] * NV)
        nfull = cnt // CH

        def chunk(k, carry):
            pltpu.sync_copy(h3p.at[pl.ds(start + k * CH, CH)], rowbuf)
            return lax.fori_loop(0, CH, row_add, carry)

        carry = lax.fori_loop(0, nfull, chunk, init)
        tail_start = start + nfull * CH
        tb = jnp.minimum(tail_start, N - CH)
        shift = tail_start - tb
        ntail = cnt - nfull * CH
        pltpu.sync_copy(h3p.at[pl.ds(tb, CH)], rowbuf)
        carry = lax.fori_loop(0, CH, row_add_masked(shift, shift + ntail),
                              carry)
        return carry

    res0 = seg_reduce(start0, cnt0)
    res1 = seg_reduce(start1, cnt1)

    for row, res, cv in ((0, res0, c0), (1, res1, c1)):
        nonempty = cv > zi
        for q in range(NV):
            outbuf[row, pl.ds(q * 16, 16)] = res[q]
            outbuf[row, pl.ds(HF + q * 16, 16)] = jnp.where(
                nonempty, res[NV + q], 0.0)
        cntbuf[row, :] = cv.astype(jnp.float32)

    pltpu.sync_copy(outbuf, hg.at[pl.ds(2 * w, 2)])
    pltpu.sync_copy(cntbuf, cnts.at[pl.ds(2 * w, 2)])


_pool = pl.kernel(
    _pool_body,
    out_type=[jax.ShapeDtypeStruct((G, 2 * HF), jnp.float32),
              jax.ShapeDtypeStruct((G, 16), jnp.float32)],
    mesh=_sc_mesh(),
    scratch_types=[
        pltpu.VMEM((N,), jnp.int32),
        pltpu.VMEM((CH, HF), jnp.float32),
        pltpu.VMEM((2, 2 * HF), jnp.float32),
        pltpu.VMEM((2, 16), jnp.float32),
    ],
    compiler_params=pltpu.CompilerParams(use_tc_tiling_on_sc=False,
                                         needs_layout_passes=False),
)


RB = 1000
NRB = N // RB


def _t1_body(x_ref, w_ref, outl, outr):
    acc = jnp.dot(x_ref[:], w_ref[:], preferred_element_type=jnp.float32)
    outl[:] = acc[:, :H]
    outr[:] = acc[:, H:]


def _t1(x, wcat):
    return pl.pallas_call(
        _t1_body,
        grid=(NRB,),
        in_specs=[
            pl.BlockSpec((RB, IN), lambda i: (i, 0)),
            pl.BlockSpec((IN, 2 * H), lambda i: (0, 0)),
        ],
        out_specs=[
            pl.BlockSpec((RB, H), lambda i: (i, 0)),
            pl.BlockSpec((RB, H), lambda i: (i, 0)),
        ],
        out_shape=[
            jax.ShapeDtypeStruct((N, H), jnp.float32),
            jax.ShapeDtypeStruct((N, H), jnp.float32),
        ],
    )(x, wcat)


def _deginv(d0, d1):
    deg = d0[:, 0:1] + d1[:, 0:1]
    return 1.0 / jnp.maximum(deg, 1.0)


def _t2_body(a0, a1, d0, d1, xr1, bl1, w2, bl2, h1_out, hr2_out):
    mean = (a0[:] + a1[:]) * _deginv(d0[:], d1[:])
    h1 = _leaky(mean + bl1[:] + xr1[:])
    h1_out[:] = h1
    hr2_out[:] = jnp.dot(h1, w2[:], preferred_element_type=jnp.float32) + bl2[:]


def _t2(a0, a1, d0, d1, xr1, bl1, w2, bl2):
    row = lambda i: (i, 0)
    zero = lambda i: (0, 0)
    return pl.pallas_call(
        _t2_body,
        grid=(NRB,),
        in_specs=[
            pl.BlockSpec((RB, H), row),
            pl.BlockSpec((RB, H), row),
            pl.BlockSpec((RB, 8), row),
            pl.BlockSpec((RB, 8), row),
            pl.BlockSpec((RB, H), row),
            pl.BlockSpec((1, H), zero),
            pl.BlockSpec((H, 2 * H), zero),
            pl.BlockSpec((1, 2 * H), zero),
        ],
        out_specs=[
            pl.BlockSpec((RB, H), row),
            pl.BlockSpec((RB, 2 * H), row),
        ],
        out_shape=[
            jax.ShapeDtypeStruct((N, H), jnp.float32),
            jax.ShapeDtypeStruct((N, 2 * H), jnp.float32),
        ],
    )(a0, a1, d0, d1, xr1, bl1, w2, bl2)


def _t3_body(a0, a1, d0, d1, hr2, wl2, wr3, bl3, h2a_out, h2b_out, hr3_out):
    mean = (a0[:] + a1[:]) * _deginv(d0[:], d1[:])
    h2 = _leaky(jnp.dot(mean, wl2[:], preferred_element_type=jnp.float32)
                + hr2[:])
    h2a_out[:] = h2[:, :H]
    h2b_out[:] = h2[:, H:]
    hr3_out[:] = jnp.dot(h2, wr3[:], preferred_element_type=jnp.float32) + bl3[:]


def _t3(a0, a1, d0, d1, hr2, wl2, wr3, bl3):
    row = lambda i: (i, 0)
    zero = lambda i: (0, 0)
    return pl.pallas_call(
        _t3_body,
        grid=(NRB,),
        in_specs=[
            pl.BlockSpec((RB, H), row),
            pl.BlockSpec((RB, H), row),
            pl.BlockSpec((RB, 8), row),
            pl.BlockSpec((RB, 8), row),
            pl.BlockSpec((RB, 2 * H), row),
            pl.BlockSpec((H, 2 * H), zero),
            pl.BlockSpec((2 * H, 4 * H), zero),
            pl.BlockSpec((1, 4 * H), zero),
        ],
        out_specs=[
            pl.BlockSpec((RB, H), row),
            pl.BlockSpec((RB, H), row),
            pl.BlockSpec((RB, 4 * H), row),
        ],
        out_shape=[
            jax.ShapeDtypeStruct((N, H), jnp.float32),
            jax.ShapeDtypeStruct((N, H), jnp.float32),
            jax.ShapeDtypeStruct((N, 4 * H), jnp.float32),
        ],
    )(a0, a1, d0, d1, hr2, wl2, wr3, bl3)


def _t4_body(a0, a1, a2, a3, d0, d1, hr3, wl3, h3_out):
    deginv = _deginv(d0[:], d1[:])
    mean = jnp.concatenate([a0[:] + a1[:], a2[:] + a3[:]], axis=1) * deginv
    h3_out[:] = _leaky(
        jnp.dot(mean, wl3[:], preferred_element_type=jnp.float32) + hr3[:])


def _t4(a0, a1, a2, a3, d0, d1, hr3, wl3):
    row = lambda i: (i, 0)
    zero = lambda i: (0, 0)
    return pl.pallas_call(
        _t4_body,
        grid=(NRB,),
        in_specs=[
            pl.BlockSpec((RB, H), row),
            pl.BlockSpec((RB, H), row),
            pl.BlockSpec((RB, H), row),
            pl.BlockSpec((RB, H), row),
            pl.BlockSpec((RB, 8), row),
            pl.BlockSpec((RB, 8), row),
            pl.BlockSpec((RB, 4 * H), row),
            pl.BlockSpec((2 * H, 4 * H), zero),
        ],
        out_specs=pl.BlockSpec((RB, 4 * H), row),
        out_shape=jax.ShapeDtypeStruct((N, 4 * H), jnp.float32),
    )(a0, a1, a2, a3, d0, d1, hr3, wl3)


def _t5_body(hg, cnts, wc1, bc1, wc2, bc2, wc3, bc3, out):
    inv = 1.0 / jnp.maximum(cnts[:, 0:1], 1.0)
    hgf = jnp.concatenate([hg[:, :HF] * inv, hg[:, HF:]], axis=1)
    z = _leaky(jnp.dot(hgf, wc1[:], preferred_element_type=jnp.float32)
               + bc1[:])
    z = _leaky(jnp.dot(z, wc2[:], preferred_element_type=jnp.float32)
               + bc2[:])
    out[:] = jnp.dot(z, wc3[:], preferred_element_type=jnp.float32) + bc3[:]


def _t5(hg, cnts, wc1, bc1, wc2, bc2, wc3p, bc3p):
    return pl.pallas_call(
        _t5_body,
        out_shape=jax.ShapeDtypeStruct((G, 128), jnp.float32),
    )(hg, cnts, wc1, bc1, wc2, bc2, wc3p, bc3p)


_agg_l = _make_agg(H)
_deg = _make_deg()


def kernel(x, edge_index, batch, Wl1, bl1, Wr1, Wl2, bl2, Wr2,
           Wl3, bl3, Wr3, Wc1, bc1, Wc2, bc2, Wc3, bc3):
    src = edge_index[0].astype(jnp.int32)
    dst = edge_index[1].astype(jnp.int32).reshape(NW, NBLK, BLK)
    batch = batch.astype(jnp.int32)

    z64 = jnp.zeros((RPW, H), jnp.float32)
    z16 = jnp.zeros((RPW, 8), jnp.float32)
    ones = jnp.ones((BLK, 8), jnp.float32)

    d0, d1 = _deg(dst, z16, ones)

    xl1, xr1 = _t1(x, jnp.concatenate([Wl1, Wr1], axis=1))
    a0, a1 = _agg_l(xl1, src, dst, z64)
    h1, hr2 = _t2(a0, a1, d0, d1, xr1, bl1.reshape(1, H),
                  Wr2, bl2.reshape(1, 2 * H))

    b0, b1 = _agg_l(h1, src, dst, z64)
    h2a, h2b, hr3 = _t3(b0, b1, d0, d1, hr2, Wl2, Wr3, bl3.reshape(1, 4 * H))

    c0a, c1a = _agg_l(h2a, src, dst, z64)
    c0b, c1b = _agg_l(h2b, src, dst, z64)
    h3 = _t4(c0a, c1a, c0b, c1b, d0, d1, hr3, Wl3)

    hg, cnts = _pool(h3, batch)
    wc3p = jnp.pad(Wc3, ((0, 0), (0, 127)))
    bc3p = jnp.pad(bc3, (0, 127)).reshape(1, 128)
    out = _t5(hg, cnts, Wc1, bc1.reshape(1, 2 * H), Wc2, bc2.reshape(1, H),
              wc3p, bc3p)
    return out[:, 0]
